# baseline (speedup 1.0000x reference)
import functools

import jax
import jax.numpy as jnp
from jax import lax
from jax.experimental import pallas as pl
from jax.experimental.pallas import tpu as pltpu
from jax.experimental.pallas import tpu_sc as plsc

EPS = 1e-08

_NC = 2
_NS = 16
_NW = _NC * _NS

_CH = 80


def _sc_gather(idx_flat, table):
    r_total = idx_flat.shape[0]
    v, d = table.shape
    rows_per_w = r_total // _NW
    chunks_per_w = rows_per_w // _CH

    mesh = plsc.VectorSubcoreMesh(
        core_axis_name="c", subcore_axis_name="s",
        num_cores=_NC, num_subcores=_NS)

    nbuf = 4
    look = 2

    @functools.partial(
        pl.kernel,
        mesh=mesh,
        out_type=jax.ShapeDtypeStruct((r_total, d), jnp.float32),
        scratch_types=[
            pltpu.VMEM((rows_per_w,), jnp.int32),
            pltpu.VMEM((nbuf, _CH, d), jnp.float32),
            pltpu.SemaphoreType.DMA,
            pltpu.SemaphoreType.DMA,
        ],
        compiler_params=pltpu.CompilerParams(use_tc_tiling_on_sc=True),
    )
    def gather_kernel(idx_hbm, table_hbm, out_hbm, idx_v, rows_v, sem_g,
                      sem_w):
        wid = lax.axis_index("s") * _NC + lax.axis_index("c")
        row0 = wid * rows_per_w
        pltpu.sync_copy(idx_hbm.at[pl.ds(row0, rows_per_w)], idx_v)

        def start_g(c):
            idx_c = idx_v.at[pl.ds(c * _CH, _CH)]
            pltpu.async_copy(table_hbm.at[idx_c], rows_v.at[c % nbuf], sem_g)

        def wait_g(c):
            idx_c = idx_v.at[pl.ds(c * _CH, _CH)]
            pltpu.make_async_copy(
                table_hbm.at[idx_c], rows_v.at[c % nbuf], sem_g).wait()

        def start_w(c):
            pltpu.async_copy(rows_v.at[c % nbuf],
                             out_hbm.at[pl.ds(row0 + c * _CH, _CH)], sem_w)

        def wait_w(c):
            pltpu.make_async_copy(
                rows_v.at[c % nbuf],
                out_hbm.at[pl.ds(row0 + c * _CH, _CH)], sem_w).wait()

        for c in range(look):
            start_g(c)

        def body(c, carry):
            wait_g(c)
            start_w(c)

            @pl.when(c + look < chunks_per_w)
            def _():
                @pl.when(c >= look)
                def _():
                    wait_w(c - look)
                start_g(c + look)

            return carry

        lax.fori_loop(0, chunks_per_w, body, 0, unroll=False)
        for c in range(chunks_per_w - nbuf, chunks_per_w):
            wait_w(c)

    return gather_kernel(idx_flat, table)


def _tc_expand_body(dst_ref, src_ref, ea_ref, out_ref):
    src = src_ref[...]
    dst = dst_ref[...]
    ea = ea_ref[...]
    a = ea.shape[1]
    d = src.shape[1]
    diff = src - dst
    nsq = jnp.sum(diff * diff, axis=1, keepdims=True)
    norm = jnp.sqrt(nsq)
    denom = norm + EPS
    inv = 1.0 / denom
    unit = diff * inv
    ea_inv = 1.0 / (ea + EPS)
    o = 0
    out_ref[:, o:o + a] = ea; o += a
    out_ref[:, o:o + d] = src; o += d
    out_ref[:, o:o + d] = dst; o += d
    out_ref[:, o:o + d] = diff; o += d
    out_ref[:, o:o + d] = unit; o += d
    out_ref[:, o:o + 1] = norm; o += 1
    out_ref[:, o:o + 1] = inv; o += 1
    out_ref[:, o:o + 1] = nsq; o += 1
    out_ref[:, o:o + 1] = inv * inv; o += 1
    out_ref[:, o:o + a] = ea_inv; o += a
    out_ref[:, o:o + a] = ea * ea; o += a
    out_ref[:, o:o + a] = ea_inv * ea_inv; o += a


def _tc_expand(gathered, edge_attr, block_rows=512):
    r, d = gathered.shape
    e, a = edge_attr.shape
    width = a * 4 + d * 4 + 4
    n_blocks = e // block_rows
    grid = (n_blocks,)
    return pl.pallas_call(
        _tc_expand_body,
        grid=grid,
        in_specs=[
            pl.BlockSpec((block_rows, d), lambda i: (i, 0)),
            pl.BlockSpec((block_rows, d), lambda i: (n_blocks + i, 0)),
            pl.BlockSpec((block_rows, a), lambda i: (i, 0)),
        ],
        out_specs=pl.BlockSpec((block_rows, width), lambda i: (i, 0)),
        out_shape=jax.ShapeDtypeStruct((e, width), jnp.float32),
        compiler_params=pltpu.CompilerParams(
            dimension_semantics=("arbitrary",),
        ),
    )(gathered, gathered, edge_attr)


def kernel(node_feat, edge_attr, edge_index):
    e = edge_index.shape[1]
    idx_flat = edge_index.reshape(-1)
    gathered = _sc_gather(idx_flat, node_feat)
    return _tc_expand(gathered, edge_attr)


# speedup vs baseline: 1.9935x; 1.9935x over previous
import functools

import jax
import jax.numpy as jnp
from jax import lax
from jax.experimental import pallas as pl
from jax.experimental.pallas import tpu as pltpu
from jax.experimental.pallas import tpu_sc as plsc

EPS = 1e-08

_NC = 2
_NS = 16
_NW = _NC * _NS

_CH = 80


def _sc_gather(idx_flat, table):
    r_total = idx_flat.shape[0]
    v, d = table.shape
    rows_per_w = r_total // _NW
    chunks_per_w = rows_per_w // _CH

    mesh = plsc.VectorSubcoreMesh(
        core_axis_name="c", subcore_axis_name="s",
        num_cores=_NC, num_subcores=_NS)

    nbuf = 4
    look = 2

    @functools.partial(
        pl.kernel,
        mesh=mesh,
        out_type=jax.ShapeDtypeStruct((r_total, d), jnp.float32),
        scratch_types=[
            pltpu.VMEM((rows_per_w,), jnp.int32),
            pltpu.VMEM((nbuf, _CH, d), jnp.float32),
            pltpu.SemaphoreType.DMA,
            pltpu.SemaphoreType.DMA,
        ],
        compiler_params=pltpu.CompilerParams(use_tc_tiling_on_sc=True),
    )
    def gather_kernel(idx_hbm, table_hbm, out_hbm, idx_v, rows_v, sem_g,
                      sem_w):
        wid = lax.axis_index("s") * _NC + lax.axis_index("c")
        row0 = wid * rows_per_w
        pltpu.sync_copy(idx_hbm.at[pl.ds(row0, rows_per_w)], idx_v)

        def start_g(c):
            idx_c = idx_v.at[pl.ds(c * _CH, _CH)]
            pltpu.async_copy(table_hbm.at[idx_c], rows_v.at[c % nbuf], sem_g)

        def wait_g(c):
            idx_c = idx_v.at[pl.ds(c * _CH, _CH)]
            pltpu.make_async_copy(
                table_hbm.at[idx_c], rows_v.at[c % nbuf], sem_g).wait()

        def start_w(c):
            pltpu.async_copy(rows_v.at[c % nbuf],
                             out_hbm.at[pl.ds(row0 + c * _CH, _CH)], sem_w)

        def wait_w(c):
            pltpu.make_async_copy(
                rows_v.at[c % nbuf],
                out_hbm.at[pl.ds(row0 + c * _CH, _CH)], sem_w).wait()

        for c in range(look):
            start_g(c)

        def body(c, carry):
            wait_g(c)
            start_w(c)

            @pl.when(c + look < chunks_per_w)
            def _():
                @pl.when(c >= look)
                def _():
                    wait_w(c - look)
                start_g(c + look)

            return carry

        lax.fori_loop(0, chunks_per_w, body, 0, unroll=False)
        for c in range(chunks_per_w - nbuf, chunks_per_w):
            wait_w(c)

    return gather_kernel(idx_flat, table)


def _tc_expand_body(dst_ref, src_ref, eat_ref, out_ref):
    src = jnp.transpose(src_ref[...], (1, 0))
    dst = jnp.transpose(dst_ref[...], (1, 0))
    ea = eat_ref[...]
    a = ea.shape[0]
    d = src.shape[0]
    diff = src - dst
    nsq = jnp.sum(diff * diff, axis=0, keepdims=True)
    norm = jnp.sqrt(nsq)
    inv = 1.0 / (norm + EPS)
    unit = diff * inv
    ea_inv = 1.0 / (ea + EPS)
    o = 0
    out_ref[o:o + a, :] = ea; o += a
    out_ref[o:o + d, :] = src; o += d
    out_ref[o:o + d, :] = dst; o += d
    out_ref[o:o + d, :] = diff; o += d
    out_ref[o:o + d, :] = unit; o += d
    out_ref[o:o + 1, :] = norm; o += 1
    out_ref[o:o + 1, :] = inv; o += 1
    out_ref[o:o + 1, :] = nsq; o += 1
    out_ref[o:o + 1, :] = inv * inv; o += 1
    out_ref[o:o + a, :] = ea_inv; o += a
    out_ref[o:o + a, :] = ea * ea; o += a
    out_ref[o:o + a, :] = ea_inv * ea_inv; o += a


def _tc_expand(gathered, edge_attr_t, block_rows=512):
    r, d = gathered.shape
    a, e = edge_attr_t.shape
    width = a * 4 + d * 4 + 4
    n_blocks = e // block_rows
    grid = (n_blocks,)
    return pl.pallas_call(
        _tc_expand_body,
        grid=grid,
        in_specs=[
            pl.BlockSpec((block_rows, d), lambda i: (i, 0)),
            pl.BlockSpec((block_rows, d), lambda i: (n_blocks + i, 0)),
            pl.BlockSpec((a, block_rows), lambda i: (0, i)),
        ],
        out_specs=pl.BlockSpec((width, block_rows), lambda i: (0, i)),
        out_shape=jax.ShapeDtypeStruct((width, e), jnp.float32),
        compiler_params=pltpu.CompilerParams(
            dimension_semantics=("arbitrary",),
        ),
    )(gathered, gathered, edge_attr_t)


def kernel(node_feat, edge_attr, edge_index):
    e = edge_index.shape[1]
    idx_flat = edge_index.reshape(-1)
    gathered = _sc_gather(idx_flat, node_feat)
    out_t = _tc_expand(gathered, edge_attr.T)
    return out_t.T


# speedup vs baseline: 2.1425x; 1.0747x over previous
import functools

import jax
import jax.numpy as jnp
from jax import lax
from jax.experimental import pallas as pl
from jax.experimental.pallas import tpu as pltpu
from jax.experimental.pallas import tpu_sc as plsc

EPS = 1e-08

_NC = 2
_NS = 16
_NW = _NC * _NS

def _pick_chunk(rows_per_w):
    for ch in range(128, 0, -8):
        if rows_per_w % ch == 0:
            return ch
    raise ValueError(rows_per_w)


def _sc_gather(idx_flat, table):
    r_total = idx_flat.shape[0]
    v, d = table.shape
    rows_per_w = r_total // _NW
    ch = _pick_chunk(rows_per_w)
    chunks_per_w = rows_per_w // ch

    mesh = plsc.VectorSubcoreMesh(
        core_axis_name="c", subcore_axis_name="s",
        num_cores=_NC, num_subcores=_NS)

    nbuf = 4
    look = 2

    @functools.partial(
        pl.kernel,
        mesh=mesh,
        out_type=jax.ShapeDtypeStruct((r_total, d), jnp.float32),
        scratch_types=[
            pltpu.VMEM((rows_per_w,), jnp.int32),
            pltpu.VMEM((nbuf, ch, d), jnp.float32),
            pltpu.SemaphoreType.DMA,
            pltpu.SemaphoreType.DMA,
        ],
        compiler_params=pltpu.CompilerParams(use_tc_tiling_on_sc=True),
    )
    def gather_kernel(idx_hbm, table_hbm, out_hbm, idx_v, rows_v, sem_g,
                      sem_w):
        wid = lax.axis_index("s") * _NC + lax.axis_index("c")
        row0 = wid * rows_per_w
        pltpu.sync_copy(idx_hbm.at[pl.ds(row0, rows_per_w)], idx_v)

        def start_g(c):
            idx_c = idx_v.at[pl.ds(c * ch, ch)]
            pltpu.async_copy(table_hbm.at[idx_c], rows_v.at[c % nbuf], sem_g)

        def wait_g(c):
            idx_c = idx_v.at[pl.ds(c * ch, ch)]
            pltpu.make_async_copy(
                table_hbm.at[idx_c], rows_v.at[c % nbuf], sem_g).wait()

        def start_w(c):
            pltpu.async_copy(rows_v.at[c % nbuf],
                             out_hbm.at[pl.ds(row0 + c * ch, ch)], sem_w)

        def wait_w(c):
            pltpu.make_async_copy(
                rows_v.at[c % nbuf],
                out_hbm.at[pl.ds(row0 + c * ch, ch)], sem_w).wait()

        for c in range(look):
            start_g(c)

        def body(c, carry):
            wait_g(c)
            start_w(c)

            @pl.when(c + look < chunks_per_w)
            def _():
                @pl.when(c >= look)
                def _():
                    wait_w(c - look)
                start_g(c + look)

            return carry

        lax.fori_loop(0, chunks_per_w, body, 0, unroll=False)
        for c in range(chunks_per_w - nbuf, chunks_per_w):
            wait_w(c)

    return gather_kernel(idx_flat, table)


def _tc_expand_body(dst_ref, src_ref, eat_ref, out_ref):
    src = jnp.transpose(src_ref[...], (1, 0))
    dst = jnp.transpose(dst_ref[...], (1, 0))
    ea = eat_ref[...]
    a = ea.shape[0]
    d = src.shape[0]
    diff = src - dst
    nsq = jnp.sum(diff * diff, axis=0, keepdims=True)
    norm = jnp.sqrt(nsq)
    inv = 1.0 / (norm + EPS)
    unit = diff * inv
    ea_inv = 1.0 / (ea + EPS)
    o = 0
    out_ref[o:o + a, :] = ea; o += a
    out_ref[o:o + d, :] = src; o += d
    out_ref[o:o + d, :] = dst; o += d
    out_ref[o:o + d, :] = diff; o += d
    out_ref[o:o + d, :] = unit; o += d
    out_ref[o:o + 1, :] = norm; o += 1
    out_ref[o:o + 1, :] = inv; o += 1
    out_ref[o:o + 1, :] = nsq; o += 1
    out_ref[o:o + 1, :] = inv * inv; o += 1
    out_ref[o:o + a, :] = ea_inv; o += a
    out_ref[o:o + a, :] = ea * ea; o += a
    out_ref[o:o + a, :] = ea_inv * ea_inv; o += a


def _alias_body(buf_ref, dst_ref, src_ref, eat_ref, out_ref):
    _tc_expand_body(dst_ref, src_ref, eat_ref, out_ref)


_BT = 512


def _tc_expand_slice(prev_buf, gathered_s, edge_attr_t, s, n_slices):
    r, d = gathered_s.shape
    a, e = edge_attr_t.shape
    width = a * 4 + d * 4 + 4
    es = e // n_slices
    nb = es // _BT
    col0 = s * nb

    common = dict(
        grid=(nb,),
        out_shape=jax.ShapeDtypeStruct((width, e), jnp.float32),
        compiler_params=pltpu.CompilerParams(
            dimension_semantics=("arbitrary",),
        ),
    )
    gspecs = [
        pl.BlockSpec((_BT, d), lambda i: (i, 0)),
        pl.BlockSpec((_BT, d), lambda i: (nb + i, 0)),
    ]
    ea_spec = pl.BlockSpec((a, _BT), lambda i: (0, col0 + i))
    out_spec = pl.BlockSpec((width, _BT), lambda i: (0, col0 + i))
    if prev_buf is None:
        return pl.pallas_call(
            _tc_expand_body,
            in_specs=gspecs + [ea_spec],
            out_specs=out_spec,
            **common,
        )(gathered_s, gathered_s, edge_attr_t)
    return pl.pallas_call(
        _alias_body,
        in_specs=[pl.BlockSpec((8, _BT), lambda i: (0, 0))] + gspecs + [ea_spec],
        out_specs=out_spec,
        input_output_aliases={0: 0},
        **common,
    )(prev_buf, gathered_s, gathered_s, edge_attr_t)


_NSLICES = 4


def kernel(node_feat, edge_attr, edge_index):
    e = edge_index.shape[1]
    es = e // _NSLICES
    ea_t = edge_attr.T
    gathered = []
    for s in range(_NSLICES):
        idx_s = edge_index[:, s * es:(s + 1) * es].reshape(-1)
        gathered.append(_sc_gather(idx_s, node_feat))
    buf = None
    for s in range(_NSLICES):
        buf = _tc_expand_slice(buf, gathered[s], ea_t, s, _NSLICES)
    return buf.T


# speedup vs baseline: 2.3099x; 1.0781x over previous
import functools

import jax
import jax.numpy as jnp
from jax import lax
from jax.experimental import pallas as pl
from jax.experimental.pallas import tpu as pltpu
from jax.experimental.pallas import tpu_sc as plsc

EPS = 1e-08

_NC = 2
_NS = 16
_NW = _NC * _NS

def _pick_chunk(rows_per_w):
    for ch in range(128, 0, -8):
        if rows_per_w % ch == 0:
            return ch
    raise ValueError(rows_per_w)


def _sc_gather(idx_flat, table):
    r_total = idx_flat.shape[0]
    v, d = table.shape
    rows_per_w = r_total // _NW
    ch = _pick_chunk(rows_per_w)
    chunks_per_w = rows_per_w // ch

    mesh = plsc.VectorSubcoreMesh(
        core_axis_name="c", subcore_axis_name="s",
        num_cores=_NC, num_subcores=_NS)

    nbuf = 4
    look = 2

    @functools.partial(
        pl.kernel,
        mesh=mesh,
        out_type=jax.ShapeDtypeStruct((r_total, d), jnp.float32),
        scratch_types=[
            pltpu.VMEM((rows_per_w,), jnp.int32),
            pltpu.VMEM((nbuf, ch, d), jnp.float32),
            pltpu.SemaphoreType.DMA,
            pltpu.SemaphoreType.DMA,
        ],
        compiler_params=pltpu.CompilerParams(use_tc_tiling_on_sc=True),
    )
    def gather_kernel(idx_hbm, table_hbm, out_hbm, idx_v, rows_v, sem_g,
                      sem_w):
        wid = lax.axis_index("s") * _NC + lax.axis_index("c")
        row0 = wid * rows_per_w
        pltpu.sync_copy(idx_hbm.at[pl.ds(row0, rows_per_w)], idx_v)

        def start_g(c):
            idx_c = idx_v.at[pl.ds(c * ch, ch)]
            pltpu.async_copy(table_hbm.at[idx_c], rows_v.at[c % nbuf], sem_g)

        def wait_g(c):
            idx_c = idx_v.at[pl.ds(c * ch, ch)]
            pltpu.make_async_copy(
                table_hbm.at[idx_c], rows_v.at[c % nbuf], sem_g).wait()

        def start_w(c):
            pltpu.async_copy(rows_v.at[c % nbuf],
                             out_hbm.at[pl.ds(row0 + c * ch, ch)], sem_w)

        def wait_w(c):
            pltpu.make_async_copy(
                rows_v.at[c % nbuf],
                out_hbm.at[pl.ds(row0 + c * ch, ch)], sem_w).wait()

        for c in range(look):
            start_g(c)

        def body(c, carry):
            wait_g(c)
            start_w(c)

            @pl.when(c + look < chunks_per_w)
            def _():
                @pl.when(c >= look)
                def _():
                    wait_w(c - look)
                start_g(c + look)

            return carry

        lax.fori_loop(0, chunks_per_w, body, 0, unroll=False)
        for c in range(chunks_per_w - nbuf, chunks_per_w):
            wait_w(c)

    return gather_kernel(idx_flat, table)


def _tc_expand_body(dst_ref, src_ref, eat_ref, out_ref):
    src = jnp.transpose(src_ref[...], (1, 0))
    dst = jnp.transpose(dst_ref[...], (1, 0))
    ea = eat_ref[...]
    a = ea.shape[0]
    d = src.shape[0]
    diff = src - dst
    nsq = jnp.sum(diff * diff, axis=0, keepdims=True)
    norm = jnp.sqrt(nsq)
    inv = 1.0 / (norm + EPS)
    unit = diff * inv
    ea_inv = 1.0 / (ea + EPS)
    o = 0
    out_ref[o:o + a, :] = ea; o += a
    out_ref[o:o + d, :] = src; o += d
    out_ref[o:o + d, :] = dst; o += d
    out_ref[o:o + d, :] = diff; o += d
    out_ref[o:o + d, :] = unit; o += d
    out_ref[o:o + 1, :] = norm; o += 1
    out_ref[o:o + 1, :] = inv; o += 1
    out_ref[o:o + 1, :] = nsq; o += 1
    out_ref[o:o + 1, :] = inv * inv; o += 1
    out_ref[o:o + a, :] = ea_inv; o += a
    out_ref[o:o + a, :] = ea * ea; o += a
    out_ref[o:o + a, :] = ea_inv * ea_inv; o += a


def _alias_body(buf_ref, dst_ref, src_ref, eat_ref, out_ref):
    _tc_expand_body(dst_ref, src_ref, eat_ref, out_ref)


_BT = 640


def _tc_expand_slice(prev_buf, gathered_s, edge_attr_t, s, n_slices):
    r, d = gathered_s.shape
    a, e = edge_attr_t.shape
    width = a * 4 + d * 4 + 4
    es = e // n_slices
    nb = es // _BT
    col0 = s * nb

    common = dict(
        grid=(nb,),
        out_shape=jax.ShapeDtypeStruct((width, e), jnp.float32),
        compiler_params=pltpu.CompilerParams(
            dimension_semantics=("arbitrary",),
        ),
    )
    gspecs = [
        pl.BlockSpec((_BT, d), lambda i: (i, 0)),
        pl.BlockSpec((_BT, d), lambda i: (nb + i, 0)),
    ]
    ea_spec = pl.BlockSpec((a, _BT), lambda i: (0, col0 + i))
    out_spec = pl.BlockSpec((width, _BT), lambda i: (0, col0 + i))
    if prev_buf is None:
        return pl.pallas_call(
            _tc_expand_body,
            in_specs=gspecs + [ea_spec],
            out_specs=out_spec,
            **common,
        )(gathered_s, gathered_s, edge_attr_t)
    return pl.pallas_call(
        _alias_body,
        in_specs=[pl.BlockSpec((8, _BT), lambda i: (0, 0))] + gspecs + [ea_spec],
        out_specs=out_spec,
        input_output_aliases={0: 0},
        **common,
    )(prev_buf, gathered_s, gathered_s, edge_attr_t)


_NSLICES = 4


def kernel(node_feat, edge_attr, edge_index):
    e = edge_index.shape[1]
    es = e // _NSLICES
    ea_t = edge_attr.T
    gathered = []
    for s in range(_NSLICES):
        idx_s = edge_index[:, s * es:(s + 1) * es].reshape(-1)
        gathered.append(_sc_gather(idx_s, node_feat))
    buf = None
    for s in range(_NSLICES):
        buf = _tc_expand_slice(buf, gathered[s], ea_t, s, _NSLICES)
    return buf.T


# speedup vs baseline: 2.7807x; 1.2038x over previous
import functools

import jax
import jax.numpy as jnp
from jax import lax
from jax.experimental import pallas as pl
from jax.experimental.pallas import tpu as pltpu
from jax.experimental.pallas import tpu_sc as plsc

EPS = 1e-08

_NC = 2
_NS = 16
_NW = _NC * _NS

def _pick_chunk(rows_per_w):
    for ch in range(128, 0, -8):
        if rows_per_w % ch == 0:
            return ch
    raise ValueError(rows_per_w)


def _sc_gather(idx_flat, table):
    r_total = idx_flat.shape[0]
    v, d = table.shape
    rows_per_w = r_total // _NW
    ch = _pick_chunk(rows_per_w)
    chunks_per_w = rows_per_w // ch

    mesh = plsc.VectorSubcoreMesh(
        core_axis_name="c", subcore_axis_name="s",
        num_cores=_NC, num_subcores=_NS)

    nbuf = 4
    look = 2

    @functools.partial(
        pl.kernel,
        mesh=mesh,
        out_type=jax.ShapeDtypeStruct((r_total, d), jnp.float32),
        scratch_types=[
            pltpu.VMEM((rows_per_w,), jnp.int32),
            pltpu.VMEM((nbuf, ch, d), jnp.float32),
            pltpu.SemaphoreType.DMA,
            pltpu.SemaphoreType.DMA,
        ],
        compiler_params=pltpu.CompilerParams(use_tc_tiling_on_sc=True),
    )
    def gather_kernel(idx_hbm, table_hbm, out_hbm, idx_v, rows_v, sem_g,
                      sem_w):
        wid = lax.axis_index("s") * _NC + lax.axis_index("c")
        row0 = wid * rows_per_w
        pltpu.sync_copy(idx_hbm.at[pl.ds(row0, rows_per_w)], idx_v)

        def start_g(c):
            idx_c = idx_v.at[pl.ds(c * ch, ch)]
            pltpu.async_copy(table_hbm.at[idx_c], rows_v.at[c % nbuf], sem_g)

        def wait_g(c):
            idx_c = idx_v.at[pl.ds(c * ch, ch)]
            pltpu.make_async_copy(
                table_hbm.at[idx_c], rows_v.at[c % nbuf], sem_g).wait()

        def start_w(c):
            pltpu.async_copy(rows_v.at[c % nbuf],
                             out_hbm.at[pl.ds(row0 + c * ch, ch)], sem_w)

        def wait_w(c):
            pltpu.make_async_copy(
                rows_v.at[c % nbuf],
                out_hbm.at[pl.ds(row0 + c * ch, ch)], sem_w).wait()

        for c in range(look):
            start_g(c)

        def body(c, carry):
            wait_g(c)
            start_w(c)

            @pl.when(c + look < chunks_per_w)
            def _():
                @pl.when(c >= look)
                def _():
                    wait_w(c - look)
                start_g(c + look)

            return carry

        lax.fori_loop(0, chunks_per_w, body, 0, unroll=False)
        for c in range(chunks_per_w - nbuf, chunks_per_w):
            wait_w(c)

    return gather_kernel(idx_flat, table)


def _tc_expand_body(dst_ref, src_ref, eat_ref, out_ref):
    src = jnp.transpose(src_ref[...], (1, 0))
    dst = jnp.transpose(dst_ref[...], (1, 0))
    ea = eat_ref[...]
    a = ea.shape[0]
    d = src.shape[0]
    diff = src - dst
    nsq = jnp.sum(diff * diff, axis=0, keepdims=True)
    norm = jnp.sqrt(nsq)
    inv = 1.0 / (norm + EPS)
    unit = diff * inv
    ea_inv = 1.0 / (ea + EPS)
    o = 0
    out_ref[o:o + a, :] = ea; o += a
    out_ref[o:o + d, :] = src; o += d
    out_ref[o:o + d, :] = dst; o += d
    out_ref[o:o + d, :] = diff; o += d
    out_ref[o:o + d, :] = unit; o += d
    out_ref[o:o + 1, :] = norm; o += 1
    out_ref[o:o + 1, :] = inv; o += 1
    out_ref[o:o + 1, :] = nsq; o += 1
    out_ref[o:o + 1, :] = inv * inv; o += 1
    out_ref[o:o + a, :] = ea_inv; o += a
    out_ref[o:o + a, :] = ea * ea; o += a
    out_ref[o:o + a, :] = ea_inv * ea_inv; o += a


def _alias_body(buf_ref, dst_ref, src_ref, eat_ref, out_ref):
    _tc_expand_body(dst_ref, src_ref, eat_ref, out_ref)


_BT = 3200


def _tc_expand_slice(prev_buf, gathered_s, edge_attr_t, s, n_slices):
    r, d = gathered_s.shape
    a, e = edge_attr_t.shape
    width = a * 4 + d * 4 + 4
    es = e // n_slices
    nb = es // _BT
    col0 = s * nb

    common = dict(
        grid=(nb,),
        out_shape=jax.ShapeDtypeStruct((width, e), jnp.float32),
        compiler_params=pltpu.CompilerParams(
            dimension_semantics=("arbitrary",),
        ),
    )
    gspecs = [
        pl.BlockSpec((_BT, d), lambda i: (i, 0)),
        pl.BlockSpec((_BT, d), lambda i: (nb + i, 0)),
    ]
    ea_spec = pl.BlockSpec((a, _BT), lambda i: (0, col0 + i))
    out_spec = pl.BlockSpec((width, _BT), lambda i: (0, col0 + i))
    if prev_buf is None:
        return pl.pallas_call(
            _tc_expand_body,
            in_specs=gspecs + [ea_spec],
            out_specs=out_spec,
            **common,
        )(gathered_s, gathered_s, edge_attr_t)
    return pl.pallas_call(
        _alias_body,
        in_specs=[pl.BlockSpec((8, _BT), lambda i: (0, 0))] + gspecs + [ea_spec],
        out_specs=out_spec,
        input_output_aliases={0: 0},
        **common,
    )(prev_buf, gathered_s, gathered_s, edge_attr_t)


_NSLICES = 4


def kernel(node_feat, edge_attr, edge_index):
    e = edge_index.shape[1]
    es = e // _NSLICES
    ea_t = edge_attr.T
    gathered = []
    for s in range(_NSLICES):
        idx_s = edge_index[:, s * es:(s + 1) * es].reshape(-1)
        gathered.append(_sc_gather(idx_s, node_feat))
    buf = None
    for s in range(_NSLICES):
        buf = _tc_expand_slice(buf, gathered[s], ea_t, s, _NSLICES)
    return buf.T


# speedup vs baseline: 2.8685x; 1.0316x over previous
import functools

import jax
import jax.numpy as jnp
from jax import lax
from jax.experimental import pallas as pl
from jax.experimental.pallas import tpu as pltpu
from jax.experimental.pallas import tpu_sc as plsc

EPS = 1e-08

_NC = 2
_NS = 16
_NW = _NC * _NS

def _pick_chunk(rows_per_w):
    for ch in range(128, 0, -8):
        if rows_per_w % ch == 0:
            return ch
    raise ValueError(rows_per_w)


def _sc_gather(idx_flat, table):
    r_total = idx_flat.shape[0]
    v, d = table.shape
    rows_per_w = r_total // _NW
    ch = _pick_chunk(rows_per_w)
    chunks_per_w = rows_per_w // ch

    mesh = plsc.VectorSubcoreMesh(
        core_axis_name="c", subcore_axis_name="s",
        num_cores=_NC, num_subcores=_NS)

    nbuf = 4
    look = 2

    @functools.partial(
        pl.kernel,
        mesh=mesh,
        out_type=jax.ShapeDtypeStruct((r_total, d), jnp.float32),
        scratch_types=[
            pltpu.VMEM((rows_per_w,), jnp.int32),
            pltpu.VMEM((nbuf, ch, d), jnp.float32),
            pltpu.SemaphoreType.DMA,
            pltpu.SemaphoreType.DMA,
        ],
        compiler_params=pltpu.CompilerParams(use_tc_tiling_on_sc=True),
    )
    def gather_kernel(idx_hbm, table_hbm, out_hbm, idx_v, rows_v, sem_g,
                      sem_w):
        wid = lax.axis_index("s") * _NC + lax.axis_index("c")
        row0 = wid * rows_per_w
        pltpu.sync_copy(idx_hbm.at[pl.ds(row0, rows_per_w)], idx_v)

        def start_g(c):
            idx_c = idx_v.at[pl.ds(c * ch, ch)]
            pltpu.async_copy(table_hbm.at[idx_c], rows_v.at[c % nbuf], sem_g)

        def wait_g(c):
            idx_c = idx_v.at[pl.ds(c * ch, ch)]
            pltpu.make_async_copy(
                table_hbm.at[idx_c], rows_v.at[c % nbuf], sem_g).wait()

        def start_w(c):
            pltpu.async_copy(rows_v.at[c % nbuf],
                             out_hbm.at[pl.ds(row0 + c * ch, ch)], sem_w)

        def wait_w(c):
            pltpu.make_async_copy(
                rows_v.at[c % nbuf],
                out_hbm.at[pl.ds(row0 + c * ch, ch)], sem_w).wait()

        for c in range(look):
            start_g(c)

        def body(c, carry):
            wait_g(c)
            start_w(c)

            @pl.when(c + look < chunks_per_w)
            def _():
                @pl.when(c >= look)
                def _():
                    wait_w(c - look)
                start_g(c + look)

            return carry

        lax.fori_loop(0, chunks_per_w, body, 0, unroll=False)
        for c in range(chunks_per_w - nbuf, chunks_per_w):
            wait_w(c)

    return gather_kernel(idx_flat, table)


def _tc_expand_body(dst_ref, src_ref, eat_ref, out_ref):
    src = jnp.transpose(src_ref[...], (1, 0))
    dst = jnp.transpose(dst_ref[...], (1, 0))
    ea = eat_ref[...]
    a = ea.shape[0]
    d = src.shape[0]
    diff = src - dst
    nsq = jnp.sum(diff * diff, axis=0, keepdims=True)
    norm = jnp.sqrt(nsq)
    inv = 1.0 / (norm + EPS)
    unit = diff * inv
    ea_inv = 1.0 / (ea + EPS)
    o = 0
    out_ref[o:o + a, :] = ea; o += a
    out_ref[o:o + d, :] = src; o += d
    out_ref[o:o + d, :] = dst; o += d
    out_ref[o:o + d, :] = diff; o += d
    out_ref[o:o + d, :] = unit; o += d
    out_ref[o:o + 1, :] = norm; o += 1
    out_ref[o:o + 1, :] = inv; o += 1
    out_ref[o:o + 1, :] = nsq; o += 1
    out_ref[o:o + 1, :] = inv * inv; o += 1
    out_ref[o:o + a, :] = ea_inv; o += a
    out_ref[o:o + a, :] = ea * ea; o += a
    out_ref[o:o + a, :] = ea_inv * ea_inv; o += a


def _alias_body(buf_ref, dst_ref, src_ref, eat_ref, out_ref):
    _tc_expand_body(dst_ref, src_ref, eat_ref, out_ref)


_BT = 3200


def _tc_expand_slice(prev_buf, gathered_s, edge_attr_t, s, n_slices):
    r, d = gathered_s.shape
    a, e = edge_attr_t.shape
    width = a * 4 + d * 4 + 4
    es = e // n_slices
    nb = es // _BT
    col0 = s * nb

    common = dict(
        grid=(nb,),
        out_shape=jax.ShapeDtypeStruct((width, e), jnp.float32),
        compiler_params=pltpu.CompilerParams(
            dimension_semantics=("arbitrary",),
        ),
    )
    gspecs = [
        pl.BlockSpec((_BT, d), lambda i: (i, 0)),
        pl.BlockSpec((_BT, d), lambda i: (nb + i, 0)),
    ]
    ea_spec = pl.BlockSpec((a, _BT), lambda i: (0, col0 + i))
    out_spec = pl.BlockSpec((width, _BT), lambda i: (0, col0 + i))
    if prev_buf is None:
        return pl.pallas_call(
            _tc_expand_body,
            in_specs=gspecs + [ea_spec],
            out_specs=out_spec,
            **common,
        )(gathered_s, gathered_s, edge_attr_t)
    return pl.pallas_call(
        _alias_body,
        in_specs=[pl.BlockSpec((8, _BT), lambda i: (0, 0))] + gspecs + [ea_spec],
        out_specs=out_spec,
        input_output_aliases={0: 0},
        **common,
    )(prev_buf, gathered_s, gathered_s, edge_attr_t)


_NSLICES = 5


def kernel(node_feat, edge_attr, edge_index):
    e = edge_index.shape[1]
    es = e // _NSLICES
    ea_t = edge_attr.T
    gathered = []
    for s in range(_NSLICES):
        idx_s = edge_index[:, s * es:(s + 1) * es].reshape(-1)
        gathered.append(_sc_gather(idx_s, node_feat))
    buf = None
    for s in range(_NSLICES):
        buf = _tc_expand_slice(buf, gathered[s], ea_t, s, _NSLICES)
    return buf.T
